# baseline (speedup 1.0000x reference)
import functools

import jax
import jax.numpy as jnp
from jax import lax
from jax.experimental import pallas as pl
from jax.experimental.pallas import tpu as pltpu
from jax.experimental.pallas import tpu_sc as plsc

B = 4096
L = 50
DIM = 128
F = 100
VOCAB = 100000

_NC = 2
_NS = 16
_NW = _NC * _NS
_CHUNK = B // 2
_EPW = _CHUNK // _NW

_mesh = plsc.VectorSubcoreMesh(core_axis_name="c", subcore_axis_name="s")


@functools.partial(
    pl.kernel,
    mesh=_mesh,
    out_type=jax.ShapeDtypeStruct((_CHUNK, DIM), jnp.float32),
    scratch_types=[
        pltpu.VMEM((L, 128), jnp.int32),
        pltpu.VMEM((_EPW, DIM), jnp.float32),
        pltpu.SemaphoreType.DMA,
    ],
)
def _pool_sum(xT_hbm, embed_hbm, out_hbm, idx_v, acc_v, sem):
    wid = lax.axis_index("s") * _NC + lax.axis_index("c")
    base = wid * _EPW
    pltpu.sync_copy(xT_hbm.at[:, pl.ds((wid // 2) * 128, 128)], idx_v)
    half = (wid % 2) * _EPW
    pltpu.async_copy(embed_hbm.at[idx_v.at[0, pl.ds(half, _EPW)]], acc_v, sem).wait()
    copies = [
        pltpu.async_copy(
            embed_hbm.at[idx_v.at[l, pl.ds(half, _EPW)]], acc_v, sem, add=True
        )
        for l in range(1, L)
    ]
    for cp in copies:
        cp.wait()
    pltpu.sync_copy(acc_v, out_hbm.at[pl.ds(base, _EPW)])


def _h_body(msum_ref, w1_ref, b1_ref, out_ref):
    out_ref[...] = (
        jnp.dot(msum_ref[...], w1_ref[...], preferred_element_type=jnp.float32)
        * (1.0 / L)
        + b1_ref[...]
    )


_VT = 1024


def _outT_body(w2_ref, h_ref, b2_ref, out_ref):
    acc = lax.dot_general(
        w2_ref[...], h_ref[...],
        (((0,), (1,)), ((), ())),
        preferred_element_type=jnp.float32,
    )
    out_ref[...] = acc + b2_ref[...]


def _outT_body_aliased(w2_ref, h_ref, b2_ref, _prev_ref, out_ref):
    _outT_body(w2_ref, h_ref, b2_ref, out_ref)


def _half(xT, embed, W1, b1):
    msum = _pool_sum(xT, embed)
    return pl.pallas_call(
        _h_body,
        out_shape=jax.ShapeDtypeStruct((_CHUNK, F), jnp.float32),
    )(msum, W1, b1.reshape(1, F))


def kernel(x, embed, W1, b1, W2, b2):
    xT = jnp.transpose(x).astype(jnp.int32)
    h1 = _half(lax.slice(xT, (0, 0), (L, _CHUNK)), embed, W1, b1)
    h2 = _half(lax.slice(xT, (0, _CHUNK), (L, B)), embed, W1, b1)

    b2c = b2.reshape(VOCAB, 1)
    nv = pl.cdiv(VOCAB, _VT)
    grid_kw = dict(
        grid=(nv,),
        out_shape=jax.ShapeDtypeStruct((VOCAB, B), jnp.float32),
    )
    w2_spec = pl.BlockSpec((F, _VT), lambda v: (0, v))
    h_spec = pl.BlockSpec((_CHUNK, F), lambda v: (0, 0))
    b2_spec = pl.BlockSpec((_VT, 1), lambda v: (v, 0))

    outT_1 = pl.pallas_call(
        _outT_body,
        in_specs=[w2_spec, h_spec, b2_spec],
        out_specs=pl.BlockSpec((_VT, _CHUNK), lambda v: (v, 0)),
        **grid_kw,
    )(W2, h1, b2c)

    outT = pl.pallas_call(
        _outT_body_aliased,
        in_specs=[w2_spec, h_spec, b2_spec, pl.BlockSpec(memory_space=pl.ANY)],
        out_specs=pl.BlockSpec((_VT, _CHUNK), lambda v: (v, 1)),
        input_output_aliases={3: 0},
        **grid_kw,
    )(W2, h2, b2c, outT_1)
    return jnp.transpose(outT)


# speedup vs baseline: 1.0859x; 1.0859x over previous
import functools

import jax
import jax.numpy as jnp
from jax import lax
from jax.experimental import pallas as pl
from jax.experimental.pallas import tpu as pltpu
from jax.experimental.pallas import tpu_sc as plsc

B = 4096
L = 50
DIM = 128
F = 100
VOCAB = 100000

_NC = 2
_NS = 16
_NW = _NC * _NS
_EPW = B // _NW

_mesh = plsc.VectorSubcoreMesh(core_axis_name="c", subcore_axis_name="s")


@functools.partial(
    pl.kernel,
    mesh=_mesh,
    out_type=jax.ShapeDtypeStruct((B, DIM), jnp.float32),
    scratch_types=[
        pltpu.VMEM((L, _EPW), jnp.int32),
        pltpu.VMEM((_EPW, DIM), jnp.float32),
        pltpu.SemaphoreType.DMA,
    ],
)
def _pool_sum(xT_hbm, embed_hbm, out_hbm, idx_v, acc_v, sem):
    wid = lax.axis_index("s") * _NC + lax.axis_index("c")
    base = wid * _EPW
    pltpu.sync_copy(xT_hbm.at[:, pl.ds(base, _EPW)], idx_v)
    pltpu.async_copy(embed_hbm.at[idx_v.at[0]], acc_v, sem).wait()
    copies = [
        pltpu.async_copy(embed_hbm.at[idx_v.at[l]], acc_v, sem, add=True)
        for l in range(1, L)
    ]
    for cp in copies:
        cp.wait()
    pltpu.sync_copy(acc_v, out_hbm.at[pl.ds(base, _EPW)])


def _h_body(msum_ref, w1_ref, b1_ref, out_ref):
    out_ref[...] = (
        jnp.dot(msum_ref[...], w1_ref[...], preferred_element_type=jnp.float32)
        * (1.0 / L)
        + b1_ref[...]
    )


_VT = 1024


def _outT_body(w2_ref, h_ref, b2_ref, out_ref):
    acc = lax.dot_general(
        w2_ref[...], h_ref[...],
        (((0,), (1,)), ((), ())),
        preferred_element_type=jnp.float32,
    )
    out_ref[...] = acc + b2_ref[...]


def kernel(x, embed, W1, b1, W2, b2):
    xT = jnp.transpose(x).astype(jnp.int32)
    msum = _pool_sum(xT, embed)

    h = pl.pallas_call(
        _h_body,
        out_shape=jax.ShapeDtypeStruct((B, F), jnp.float32),
    )(msum, W1, b1.reshape(1, F))

    outT = pl.pallas_call(
        _outT_body,
        grid=(pl.cdiv(VOCAB, _VT),),
        in_specs=[
            pl.BlockSpec((F, _VT), lambda v: (0, v)),
            pl.BlockSpec((B, F), lambda v: (0, 0)),
            pl.BlockSpec((_VT, 1), lambda v: (v, 0)),
        ],
        out_specs=pl.BlockSpec((_VT, B), lambda v: (v, 0)),
        out_shape=jax.ShapeDtypeStruct((VOCAB, B), jnp.float32),
    )(W2, h, b2.reshape(VOCAB, 1))
    return jnp.transpose(outT)
